# baseline (speedup 1.0000x reference)
import functools

import jax
import jax.numpy as jnp
from jax import lax
from jax.experimental import pallas as pl
from jax.experimental.pallas import tpu as pltpu
from jax.experimental.pallas import tpu_sc as plsc

EMBED_DIM = 32
ROWS_PER_MEGA = 128 // EMBED_DIM
NUM_CORES = 2
NUM_SUBCORES = 16
NUM_WORKERS = NUM_CORES * NUM_SUBCORES
CHUNK = 128


@functools.lru_cache(maxsize=None)
def _make_gather(n_rows: int, vocab: int):
    per_w = n_rows // NUM_WORKERS
    n_chunks = per_w // CHUNK
    out_mega_per_chunk = CHUNK // ROWS_PER_MEGA
    mesh = plsc.VectorSubcoreMesh(
        core_axis_name="c",
        subcore_axis_name="s",
        num_cores=NUM_CORES,
        num_subcores=NUM_SUBCORES,
    )

    @functools.partial(
        pl.kernel,
        out_type=jax.ShapeDtypeStruct((n_rows // ROWS_PER_MEGA, 128), jnp.float32),
        mesh=mesh,
        scratch_types=[
            pltpu.VMEM((n_chunks, CHUNK), jnp.int32),
            pltpu.VMEM((CHUNK,), jnp.int32),
            pltpu.VMEM((CHUNK, 128), jnp.float32),
            pltpu.VMEM((out_mega_per_chunk, 128), jnp.float32),
            pltpu.SemaphoreType.DMA,
        ],
    )
    def gather_kernel(idx_hbm, table_hbm, out_hbm, idx_v, mega_v, rows_v, out_v, sem):
        wid = lax.axis_index("s") * NUM_CORES + lax.axis_index("c")
        pltpu.sync_copy(idx_hbm.at[wid], idx_v)
        mega_base = wid * (per_w // ROWS_PER_MEGA)

        def chunk_body(j, carry):
            for g in range(CHUNK // 16):
                sl = pl.ds(g * 16, 16)
                mega_v[sl] = lax.shift_right_logical(idx_v[j, sl], 2)
            pltpu.async_copy(table_hbm.at[mega_v], rows_v, sem).wait()

            def grp_body(g, carry2):
                base = g * 16
                offs = (idx_v[j, pl.ds(base, 16)] & 3) * EMBED_DIM
                for lane in range(16):
                    src = offs[lane]
                    i = base + lane
                    om = g * 4 + (lane // 4)
                    dst = (lane % 4) * EMBED_DIM
                    out_v[om, pl.ds(dst, 16)] = rows_v[i, pl.ds(src, 16)]
                    out_v[om, pl.ds(dst + 16, 16)] = rows_v[i, pl.ds(src + 16, 16)]
                return carry2

            lax.fori_loop(0, CHUNK // 16, grp_body, 0)
            pltpu.sync_copy(
                out_v,
                out_hbm.at[pl.ds(mega_base + j * out_mega_per_chunk,
                                 out_mega_per_chunk)],
            )
            return carry

        lax.fori_loop(0, n_chunks, chunk_body, 0)

    return gather_kernel


def kernel(inputs, table):
    b, h = inputs.shape
    n_rows = b * h
    vocab = table.shape[0]
    idx = inputs.astype(jnp.int32).reshape(
        NUM_WORKERS, n_rows // (NUM_WORKERS * CHUNK), CHUNK
    )
    table_mega = table.reshape(vocab // ROWS_PER_MEGA, 128)
    out = _make_gather(n_rows, vocab)(idx, table_mega)
    return out.reshape(b, h, EMBED_DIM)


# speedup vs baseline: 1.2190x; 1.2190x over previous
import functools

import jax
import jax.numpy as jnp
from jax import lax
from jax.experimental import pallas as pl
from jax.experimental.pallas import tpu as pltpu
from jax.experimental.pallas import tpu_sc as plsc

EMBED_DIM = 32
ROWS_PER_MEGA = 128 // EMBED_DIM
NUM_CORES = 2
NUM_SUBCORES = 16
NUM_WORKERS = NUM_CORES * NUM_SUBCORES
CHUNK = 128


@functools.lru_cache(maxsize=None)
def _make_gather(n_h: int, n_b: int, vocab: int):
    mesh = plsc.VectorSubcoreMesh(
        core_axis_name="c",
        subcore_axis_name="s",
        num_cores=NUM_CORES,
        num_subcores=NUM_SUBCORES,
    )

    @functools.partial(
        pl.kernel,
        out_type=jax.ShapeDtypeStruct((n_h, EMBED_DIM, n_b), jnp.float32),
        mesh=mesh,
        scratch_types=[
            pltpu.VMEM((n_h, CHUNK), jnp.int32),
            pltpu.VMEM((2, CHUNK), jnp.int32),
            pltpu.VMEM((CHUNK,), jnp.int32),
            pltpu.VMEM((2, CHUNK, 128), jnp.float32),
            pltpu.VMEM((EMBED_DIM, CHUNK), jnp.float32),
            pltpu.SemaphoreType.DMA,
            pltpu.SemaphoreType.DMA,
        ],
        compiler_params=pltpu.CompilerParams(needs_layout_passes=False),
    )
    def gather_kernel(idx_hbm, table_hbm, out_hbm, idx_v, mega_v, rem_v,
                      rows_v, outt_v, gsem, osem):
        wid = lax.axis_index("s") * NUM_CORES + lax.axis_index("c")
        b0 = wid * CHUNK
        pltpu.sync_copy(idx_hbm.at[:, pl.ds(b0, CHUNK)], idx_v)

        def fire(h, buf):
            for g in range(CHUNK // 16):
                sl = pl.ds(g * 16, 16)
                mega_v[buf, sl] = lax.shift_right_logical(idx_v[h, sl], 2)
            return pltpu.async_copy(
                table_hbm.at[mega_v.at[buf]], rows_v.at[buf], gsem
            )

        fire(0, 0).wait()

        def chunk_body(h, carry):
            buf = lax.rem(h, 2)
            @pl.when(h + 1 < n_h)
            def _():
                fire(h + 1, 1 - buf)

            for g in range(CHUNK // 16):
                sl = pl.ds(g * 16, 16)
                rem_v[sl] = (idx_v[h, sl] & 3) * EMBED_DIM

            for g in range(CHUNK // 16):
                sl = pl.ds(g * 16, 16)
                rowv = lax.iota(jnp.int32, 16) + g * 16
                remg = rem_v[sl]

                def d_body(d, carry2):
                    vals = plsc.load_gather(rows_v.at[buf], [rowv, remg + d])
                    outt_v[d, sl] = vals
                    return carry2

                lax.fori_loop(0, EMBED_DIM, d_body, 0)

            pltpu.sync_copy(outt_v, out_hbm.at[h, :, pl.ds(b0, CHUNK)])

            @pl.when(h + 1 < n_h)
            def _():
                pltpu.make_async_copy(
                    table_hbm.at[mega_v.at[1 - buf]], rows_v.at[1 - buf], gsem
                ).wait()

            return carry

        lax.fori_loop(0, n_h, chunk_body, 0)

    return gather_kernel


def kernel(inputs, table):
    b, h = inputs.shape
    vocab = table.shape[0]
    idx_t = inputs.astype(jnp.int32).T
    table_mega = table.reshape(vocab // ROWS_PER_MEGA, 128)
    out_phys = _make_gather(h, b, vocab)(idx_t, table_mega)
    return jnp.transpose(out_phys, (2, 0, 1))
